# baseline (speedup 1.0000x reference)
import jax
import jax.numpy as jnp
from jax import lax
from jax.experimental import pallas as pl
from jax.experimental.pallas import tpu as pltpu

_TOP_K = 1000
_CONF = 0.01
_B = 4
_N0 = 4096
_K = 4096
_W = 64
_ROWS = 512
_NSTEP = _N0 // _ROWS
_LANES = 512


def _reduce_rows(xa, xb):
    htile = _K // 256
    vm = xa[:, :, 0:128]
    it = jnp.zeros((1, _ROWS, 128), jnp.int32)
    for t in range(1, 2 * htile):
        x = xa if t < htile else xb
        lt = t if t < htile else t - htile
        xt = x[:, :, lt * 128:(lt + 1) * 128]
        gt = xt > vm
        it = jnp.where(gt, t, it)
        vm = jnp.where(gt, xt, vm)
    m = jnp.max(vm, axis=-1)
    lane = lax.broadcasted_iota(jnp.int32, (1, _ROWS, 128), 2)
    g = (it << 7) | lane
    hit = jnp.where(vm == m[..., None], g, _K)
    mi = jnp.min(hit, axis=-1)
    return m, mi


def _wins(va, pa, vb, pb):
    return (va > vb) | ((va == vb) & (pa < pb))


def _cx(vA, pA, vB, pB):
    aw = _wins(vA, pA, vB, pB)
    w = (jnp.where(aw, vA, vB), jnp.where(aw, pA, pB))
    l = (jnp.where(aw, vB, vA), jnp.where(aw, pB, pA))
    return w, l


def _sub_stage(v, p, j, k, truncate=False):
    js = j // _LANES
    nch = v.shape[1] // js
    vout = [None] * nch
    pout = [None] * nch
    for c in range(nch // 2):
        a, b = 2 * c, 2 * c + 1
        vA, pA = v[:, a * js:(a + 1) * js], p[:, a * js:(a + 1) * js]
        vB, pB = v[:, b * js:(b + 1) * js], p[:, b * js:(b + 1) * js]
        (wv, wp), (lv, lp) = _cx(vA, pA, vB, pB)
        desc = True if k >= 4096 else ((a * js) & (k // _LANES)) == 0
        if desc:
            vout[a], pout[a], vout[b], pout[b] = wv, wp, lv, lp
        else:
            vout[a], pout[a], vout[b], pout[b] = lv, lp, wv, wp
    if truncate:
        vout, pout = vout[:nch // 2], pout[:nch // 2]
    if len(vout) == 1:
        return vout[0], pout[0]
    return jnp.concatenate(vout, axis=1), jnp.concatenate(pout, axis=1)


def _lane_chunk_stage(v, p, j, k, s_idx):
    nch = _LANES // j
    vout = [None] * nch
    pout = [None] * nch
    for c in range(nch // 2):
        a, b = 2 * c, 2 * c + 1
        vA, pA = v[..., a * j:(a + 1) * j], p[..., a * j:(a + 1) * j]
        vB, pB = v[..., b * j:(b + 1) * j], p[..., b * j:(b + 1) * j]
        (wv, wp), (lv, lp) = _cx(vA, pA, vB, pB)
        if k >= 4096:
            desc = True
        elif k >= _LANES:
            desc = (s_idx[..., :j] & (k // _LANES)) == 0
        else:
            desc = ((a * j) & k) == 0
        if desc is True:
            vout[a], pout[a], vout[b], pout[b] = wv, wp, lv, lp
        elif desc is False:
            vout[a], pout[a], vout[b], pout[b] = lv, lp, wv, wp
        else:
            vout[a] = jnp.where(desc, wv, lv)
            pout[a] = jnp.where(desc, wp, lp)
            vout[b] = jnp.where(desc, lv, wv)
            pout[b] = jnp.where(desc, lp, wp)
    return jnp.concatenate(vout, axis=-1), jnp.concatenate(pout, axis=-1)


def _roll_stage(v, p, j, k, l_idx, s_idx):
    bit_lo = (l_idx & j) == 0
    if k >= 4096:
        desc = True
    elif k >= _LANES:
        desc = (s_idx & (k // _LANES)) == 0
    else:
        desc = (l_idx & k) == 0
    vf = jnp.concatenate([v[..., j:], v[..., :j]], axis=-1)
    vb = jnp.concatenate([v[..., _LANES - j:], v[..., :_LANES - j]], axis=-1)
    pf = jnp.concatenate([p[..., j:], p[..., :j]], axis=-1)
    pb = jnp.concatenate([p[..., _LANES - j:], p[..., :_LANES - j]], axis=-1)
    pv = jnp.where(bit_lo, vf, vb)
    pp = jnp.where(bit_lo, pf, pb)
    self_wins = _wins(v, p, pv, pp)
    if desc is True:
        keep = self_wins == bit_lo
    else:
        keep = (self_wins == bit_lo) == desc
    return jnp.where(keep, v, pv), jnp.where(keep, p, pp)


def _topk_compute(m, anch, conf_ref, x0_ref, y0_ref, x1_ref, y1_ref):
    shp = m.shape
    l_idx = lax.broadcasted_iota(jnp.int32, shp, 2)
    s_idx = lax.broadcasted_iota(jnp.int32, shp, 1)
    v = jnp.where(m > _CONF, m, -jnp.inf)
    p = (((s_idx << 9) | l_idx) << 12) | anch

    def stage(v, p, j, k, truncate=False):
        if j >= _LANES:
            return _sub_stage(v, p, j, k, truncate)
        sub = v.shape[1]
        if j >= 128:
            return _lane_chunk_stage(v, p, j, k, s_idx[:, :sub])
        return _roll_stage(v, p, j, k, l_idx[:, :sub], s_idx[:, :sub])

    k = 2
    while k <= 2048:
        j = k // 2
        while j >= 1:
            v, p = stage(v, p, j, k)
            j //= 2
        k *= 2

    v, p = stage(v, p, 2048, 4096, truncate=True)
    v, p = stage(v, p, 1024, 4096, truncate=True)
    j = 512
    while j >= 1:
        v, p = stage(v, p, j, 4096)
        j //= 2

    v = jnp.concatenate([v[:, 0], v[:, 1]], axis=-1)
    p = jnp.concatenate([p[:, 0], p[:, 1]], axis=-1)
    sidx = p >> 12
    sanch = p & (_N0 - 1)
    valid = v > _CONF
    inv = jnp.float32(1.0 / (_W - 1))
    fz = jnp.float32(0.0)
    conf_ref[...] = jnp.where(valid, v, fz)[:, :_TOP_K]
    x0_ref[...] = jnp.where(valid, (sidx & (_W - 1)).astype(jnp.float32) * inv, fz)[:, :_TOP_K]
    y0_ref[...] = jnp.where(valid, ((sidx >> 6) & (_W - 1)).astype(jnp.float32) * inv, fz)[:, :_TOP_K]
    x1_ref[...] = jnp.where(valid, (sanch & (_W - 1)).astype(jnp.float32) * inv, fz)[:, :_TOP_K]
    y1_ref[...] = jnp.where(valid, (sanch >> 6).astype(jnp.float32) * inv, fz)[:, :_TOP_K]


def _fused_body(xa_ref, xb_ref, conf_ref, x0_ref, y0_ref, x1_ref, y1_ref, mv_acc, mi_acc):
    g = pl.program_id(0)
    b = g // _NSTEP
    r = g % _NSTEP
    m, mi = _reduce_rows(xa_ref[...], xb_ref[...])
    mv_acc[b, r] = m[0]
    mi_acc[b, r] = mi[0]

    @pl.when(g == _B * _NSTEP - 1)
    def _():
        _topk_compute(mv_acc[...], mi_acc[...],
                      conf_ref, x0_ref, y0_ref, x1_ref, y1_ref)


def kernel(anchor_probs):
    B, N0, K = anchor_probs.shape
    out_spec = pl.BlockSpec((_B, _TOP_K), lambda g: (0, 0))
    conf, x0, y0, x1, y1 = pl.pallas_call(
        _fused_body,
        grid=(B * _NSTEP,),
        in_specs=[
            pl.BlockSpec((1, _ROWS, K // 2), lambda g: (g // _NSTEP, g % _NSTEP, 0)),
            pl.BlockSpec((1, _ROWS, K // 2), lambda g: (g // _NSTEP, g % _NSTEP, 1)),
        ],
        out_specs=[out_spec] * 5,
        out_shape=[jax.ShapeDtypeStruct((_B, _TOP_K), jnp.float32)] * 5,
        scratch_shapes=[
            pltpu.VMEM((_B, _NSTEP, _ROWS), jnp.float32),
            pltpu.VMEM((_B, _NSTEP, _ROWS), jnp.int32),
        ],
    )(anchor_probs, anchor_probs)

    mkpts0 = jnp.stack([x0, y0], axis=-1).reshape(-1, 2)
    mkpts1 = jnp.stack([x1, y1], axis=-1).reshape(-1, 2)
    mconf = conf.reshape(-1)
    b_ids = jnp.broadcast_to(jnp.arange(B)[:, None], (B, _TOP_K)).reshape(-1)
    return (mkpts0, mkpts1, mconf, b_ids)


# speedup vs baseline: 1.0006x; 1.0006x over previous
import jax
import jax.numpy as jnp
from jax import lax
from jax.experimental import pallas as pl
from jax.experimental.pallas import tpu as pltpu

_TOP_K = 1000
_CONF = 0.01
_B = 4
_N0 = 4096
_K = 4096
_W = 64
_ROWS = 512
_NSTEP = _N0 // _ROWS
_LANES = 512


def _reduce_rows(x):
    ntile = _K // 128
    vm = x[:, :, 0:128]
    it = jnp.zeros((1, _ROWS, 128), jnp.int32)
    for t in range(1, ntile):
        xt = x[:, :, t * 128:(t + 1) * 128]
        gt = xt > vm
        it = jnp.where(gt, t, it)
        vm = jnp.where(gt, xt, vm)
    m = jnp.max(vm, axis=-1)
    lane = lax.broadcasted_iota(jnp.int32, (1, _ROWS, 128), 2)
    g = (it << 7) | lane
    hit = jnp.where(vm == m[..., None], g, _K)
    mi = jnp.min(hit, axis=-1)
    return m, mi


def _wins(va, pa, vb, pb):
    return (va > vb) | ((va == vb) & (pa < pb))


def _cx(vA, pA, vB, pB):
    aw = _wins(vA, pA, vB, pB)
    w = (jnp.where(aw, vA, vB), jnp.where(aw, pA, pB))
    l = (jnp.where(aw, vB, vA), jnp.where(aw, pB, pA))
    return w, l


def _sub_stage(v, p, j, k, truncate=False):
    js = j // _LANES
    nch = v.shape[1] // js
    vout = [None] * nch
    pout = [None] * nch
    for c in range(nch // 2):
        a, b = 2 * c, 2 * c + 1
        vA, pA = v[:, a * js:(a + 1) * js], p[:, a * js:(a + 1) * js]
        vB, pB = v[:, b * js:(b + 1) * js], p[:, b * js:(b + 1) * js]
        (wv, wp), (lv, lp) = _cx(vA, pA, vB, pB)
        desc = True if k >= 4096 else ((a * js) & (k // _LANES)) == 0
        if desc:
            vout[a], pout[a], vout[b], pout[b] = wv, wp, lv, lp
        else:
            vout[a], pout[a], vout[b], pout[b] = lv, lp, wv, wp
    if truncate:
        vout, pout = vout[:nch // 2], pout[:nch // 2]
    if len(vout) == 1:
        return vout[0], pout[0]
    return jnp.concatenate(vout, axis=1), jnp.concatenate(pout, axis=1)


def _lane_chunk_stage(v, p, j, k, s_idx):
    nch = _LANES // j
    vout = [None] * nch
    pout = [None] * nch
    for c in range(nch // 2):
        a, b = 2 * c, 2 * c + 1
        vA, pA = v[..., a * j:(a + 1) * j], p[..., a * j:(a + 1) * j]
        vB, pB = v[..., b * j:(b + 1) * j], p[..., b * j:(b + 1) * j]
        (wv, wp), (lv, lp) = _cx(vA, pA, vB, pB)
        if k >= 4096:
            desc = True
        elif k >= _LANES:
            desc = (s_idx[..., :j] & (k // _LANES)) == 0
        else:
            desc = ((a * j) & k) == 0
        if desc is True:
            vout[a], pout[a], vout[b], pout[b] = wv, wp, lv, lp
        elif desc is False:
            vout[a], pout[a], vout[b], pout[b] = lv, lp, wv, wp
        else:
            vout[a] = jnp.where(desc, wv, lv)
            pout[a] = jnp.where(desc, wp, lp)
            vout[b] = jnp.where(desc, lv, wv)
            pout[b] = jnp.where(desc, lp, wp)
    return jnp.concatenate(vout, axis=-1), jnp.concatenate(pout, axis=-1)


def _roll_stage(v, p, j, k, l_idx, s_idx):
    bit_lo = (l_idx & j) == 0
    if k >= 4096:
        desc = True
    elif k >= _LANES:
        desc = (s_idx & (k // _LANES)) == 0
    else:
        desc = (l_idx & k) == 0
    vf = jnp.concatenate([v[..., j:], v[..., :j]], axis=-1)
    vb = jnp.concatenate([v[..., _LANES - j:], v[..., :_LANES - j]], axis=-1)
    pf = jnp.concatenate([p[..., j:], p[..., :j]], axis=-1)
    pb = jnp.concatenate([p[..., _LANES - j:], p[..., :_LANES - j]], axis=-1)
    pv = jnp.where(bit_lo, vf, vb)
    pp = jnp.where(bit_lo, pf, pb)
    self_wins = _wins(v, p, pv, pp)
    if desc is True:
        keep = self_wins == bit_lo
    else:
        keep = (self_wins == bit_lo) == desc
    return jnp.where(keep, v, pv), jnp.where(keep, p, pp)


def _topk_compute(m, anch, conf_ref, x0_ref, y0_ref, x1_ref, y1_ref):
    shp = m.shape
    l_idx = lax.broadcasted_iota(jnp.int32, shp, 2)
    s_idx = lax.broadcasted_iota(jnp.int32, shp, 1)
    v = jnp.where(m > _CONF, m, -jnp.inf)
    p = (((s_idx << 9) | l_idx) << 12) | anch

    def stage(v, p, j, k, truncate=False):
        if j >= _LANES:
            return _sub_stage(v, p, j, k, truncate)
        sub = v.shape[1]
        if j >= 128:
            return _lane_chunk_stage(v, p, j, k, s_idx[:, :sub])
        return _roll_stage(v, p, j, k, l_idx[:, :sub], s_idx[:, :sub])

    k = 2
    while k <= 2048:
        j = k // 2
        while j >= 1:
            v, p = stage(v, p, j, k)
            j //= 2
        k *= 2

    v, p = stage(v, p, 2048, 4096, truncate=True)
    v, p = stage(v, p, 1024, 4096, truncate=True)
    j = 512
    while j >= 1:
        v, p = stage(v, p, j, 4096)
        j //= 2

    v = jnp.concatenate([v[:, 0], v[:, 1]], axis=-1)
    p = jnp.concatenate([p[:, 0], p[:, 1]], axis=-1)
    sidx = p >> 12
    sanch = p & (_N0 - 1)
    valid = v > _CONF
    inv = jnp.float32(1.0 / (_W - 1))
    fz = jnp.float32(0.0)
    conf_ref[...] = jnp.where(valid, v, fz)[:, :_TOP_K]
    x0_ref[...] = jnp.where(valid, (sidx & (_W - 1)).astype(jnp.float32) * inv, fz)[:, :_TOP_K]
    y0_ref[...] = jnp.where(valid, ((sidx >> 6) & (_W - 1)).astype(jnp.float32) * inv, fz)[:, :_TOP_K]
    x1_ref[...] = jnp.where(valid, (sanch & (_W - 1)).astype(jnp.float32) * inv, fz)[:, :_TOP_K]
    y1_ref[...] = jnp.where(valid, (sanch >> 6).astype(jnp.float32) * inv, fz)[:, :_TOP_K]


def _fused_body(x_ref, conf_ref, x0_ref, y0_ref, x1_ref, y1_ref, mv_acc, mi_acc):
    g = pl.program_id(0)
    b = g // _NSTEP
    r = g % _NSTEP
    m, mi = _reduce_rows(x_ref[...])
    mv_acc[b, r] = m[0]
    mi_acc[b, r] = mi[0]

    @pl.when(g == _B * _NSTEP - 1)
    def _():
        _topk_compute(mv_acc[...], mi_acc[...],
                      conf_ref, x0_ref, y0_ref, x1_ref, y1_ref)


def kernel(anchor_probs):
    B, N0, K = anchor_probs.shape
    out_spec = pl.BlockSpec((_B, _TOP_K), lambda g: (0, 0))
    conf, x0, y0, x1, y1 = pl.pallas_call(
        _fused_body,
        grid=(B * _NSTEP,),
        in_specs=[pl.BlockSpec((1, _ROWS, K), lambda g: (g // _NSTEP, g % _NSTEP, 0))],
        out_specs=[out_spec] * 5,
        out_shape=[jax.ShapeDtypeStruct((_B, _TOP_K), jnp.float32)] * 5,
        scratch_shapes=[
            pltpu.VMEM((_B, _NSTEP, _ROWS), jnp.float32),
            pltpu.VMEM((_B, _NSTEP, _ROWS), jnp.int32),
        ],
    )(anchor_probs)

    mkpts0 = jnp.stack([x0, y0], axis=-1).reshape(-1, 2)
    mkpts1 = jnp.stack([x1, y1], axis=-1).reshape(-1, 2)
    mconf = conf.reshape(-1)
    b_ids = jnp.broadcast_to(jnp.arange(B)[:, None], (B, _TOP_K)).reshape(-1)
    return (mkpts0, mkpts1, mconf, b_ids)
